# baseline (speedup 1.0000x reference)
import functools

import jax
import jax.numpy as jnp
from jax import lax
from jax.experimental import pallas as pl
from jax.experimental.pallas import tpu as pltpu
from jax.experimental.pallas import tpu_sc as plsc

_UNROLL = 8


def _make_sc_add(B, L, D):
    info = plsc.get_sparse_core_info()
    NC, NS = info.num_cores, info.num_subcores
    NW = NC * NS
    CH = L * D
    n_chunks = B // NW
    n_pairs = n_chunks // 2
    per_w = n_chunks * CH
    NV = CH // 16

    mesh = plsc.VectorSubcoreMesh(core_axis_name="c", subcore_axis_name="s")

    @functools.partial(
        pl.kernel,
        mesh=mesh,
        out_type=jax.ShapeDtypeStruct((B * CH,), jnp.float32),
        scratch_types=[
            pltpu.VMEM((CH,), jnp.float32),
            pltpu.VMEM((2, CH), jnp.float32),
            pltpu.SemaphoreType.DMA,
            pltpu.SemaphoreType.DMA,
            pltpu.SemaphoreType.DMA,
            pltpu.SemaphoreType.DMA,
        ],
    )
    def sc_add(x_hbm, pe_hbm, out_hbm, pe_v, xb, in0, in1, o0, o1):
        wid = lax.axis_index("s") * NC + lax.axis_index("c")
        base = wid * per_w
        pltpu.sync_copy(pe_hbm.at[pl.ds(0, CH)], pe_v)

        in_sems = (in0, in1)
        out_sems = (o0, o1)

        def in_copy(t, p):
            return pltpu.make_async_copy(
                x_hbm.at[pl.ds(base + t * CH, CH)], xb.at[p], in_sems[p])

        def out_copy(t, p):
            return pltpu.make_async_copy(
                xb.at[p], out_hbm.at[pl.ds(base + t * CH, CH)], out_sems[p])

        def add_pe(p):
            def body(i, carry):
                for u in range(_UNROLL):
                    s = pl.ds((i * _UNROLL + u) * 16, 16)
                    xb[p, s] = xb[p, s] + pe_v[s]
                return carry
            lax.fori_loop(0, NV // _UNROLL, body, 0)

        def pair(j, carry):
            t0 = 2 * j
            in_copy(t0, 0).wait()

            @pl.when(j > 0)
            def _():
                out_copy(t0 - 1, 1).wait()

            in_copy(t0 + 1, 1).start()
            add_pe(0)
            out_copy(t0, 0).start()
            in_copy(t0 + 1, 1).wait()
            add_pe(1)
            out_copy(t0, 0).wait()

            @pl.when(j < n_pairs - 1)
            def _():
                in_copy(t0 + 2, 0).start()

            out_copy(t0 + 1, 1).start()
            return carry

        in_copy(0, 0).start()
        lax.fori_loop(0, n_pairs, pair, 0)
        out_copy(n_chunks - 1, 1).wait()

    return sc_add


@jax.jit
def kernel(x, pos_embed):
    B, L, D = x.shape
    x_flat = x.reshape(-1)
    pe_flat = pos_embed.reshape(-1)
    out = _make_sc_add(B, L, D)(x_flat, pe_flat)
    return out.reshape(B, L, D)


# speedup vs baseline: 4.3433x; 4.3433x over previous
import functools

import jax
import jax.numpy as jnp
from jax import lax
from jax.experimental import pallas as pl
from jax.experimental.pallas import tpu as pltpu
from jax.experimental.pallas import tpu_sc as plsc

_ROWS_PER_WORKER = 8


def _make_sc_gather(max_len, L, D):
    info = plsc.get_sparse_core_info()
    NS = info.num_subcores
    n_work = L // _ROWS_PER_WORKER
    n_rounds = -(-n_work // NS)

    mesh = plsc.VectorSubcoreMesh(
        core_axis_name="c", subcore_axis_name="s", num_cores=1)

    @functools.partial(
        pl.kernel,
        mesh=mesh,
        out_type=jax.ShapeDtypeStruct((L, D), jnp.float32),
        scratch_types=[
            pltpu.VMEM((_ROWS_PER_WORKER,), jnp.int32),
            pltpu.VMEM((_ROWS_PER_WORKER, D), jnp.float32),
            pltpu.SemaphoreType.DMA,
        ],
    )
    def gather_pe(table_hbm, idx_hbm, out_hbm, idx_v, rows_v, sem):
        sid = lax.axis_index("s")
        for k in range(n_rounds):
            chunk = sid + k * NS

            @pl.when(chunk < n_work)
            def _():
                base = chunk * _ROWS_PER_WORKER
                pltpu.sync_copy(idx_hbm.at[pl.ds(base, _ROWS_PER_WORKER)], idx_v)
                pltpu.async_copy(table_hbm.at[idx_v], rows_v, sem).wait()
                pltpu.sync_copy(rows_v, out_hbm.at[pl.ds(base, _ROWS_PER_WORKER)])

    return gather_pe


def _add_pe_kernel(x_ref, pe_ref, o_ref):
    o_ref[...] = x_ref[...] + pe_ref[...]


@jax.jit
def kernel(x, pos_embed):
    B, L, D = x.shape
    max_len = pos_embed.shape[0]
    positions = jnp.arange(L, dtype=jnp.int32)

    pe = _make_sc_gather(max_len, L, D)(pos_embed, positions)

    bB = next(b for b in (128, 64, 32, 16, 8, 4, 2, 1) if B % b == 0)
    return pl.pallas_call(
        _add_pe_kernel,
        grid=(B // bB,),
        in_specs=[
            pl.BlockSpec((bB, L, D), lambda i: (i, 0, 0)),
            pl.BlockSpec((L, D), lambda i: (0, 0)),
        ],
        out_specs=pl.BlockSpec((bB, L, D), lambda i: (i, 0, 0)),
        out_shape=jax.ShapeDtypeStruct((B, L, D), x.dtype),
    )(x, pe)
